# Initial kernel scaffold; baseline (speedup 1.0000x reference)
import functools

import jax
import jax.numpy as jnp
from jax import lax
from jax.experimental import pallas as pl
from jax.experimental.pallas import tpu as pltpu
from jax.experimental.pallas import tpu_sc as plsc

NUM_USERS = 25000
NUM_ITEMS = 25000
NUM_NODES = NUM_USERS + NUM_ITEMS
EMBED_DIM = 64
NUM_EDGES = 800000
NUM_LAYERS = 3

HALF = 25000
ACC_ROWS = 25024
ROWS_PER_TILE = ACC_ROWS // 16
CHUNK = 128
NCHUNK = 392
EDGES_PER_TILE = CHUNK * NCHUNK
E_PAD = EDGES_PER_TILE * 16

_COL3 = jnp.arange(0, 48, 3, dtype=jnp.int32)


def _propagate_layer(emb, edges_flat, zeros_acc):
    mesh = plsc.VectorSubcoreMesh(core_axis_name="c", subcore_axis_name="s")

    @functools.partial(
        pl.kernel,
        mesh=mesh,
        out_type=jax.ShapeDtypeStruct((NUM_NODES, EMBED_DIM), jnp.float32),
        scratch_types=[
            pltpu.VMEM_SHARED((ACC_ROWS, EMBED_DIM), jnp.float32),
            pltpu.VMEM((384,), jnp.int32),
            pltpu.VMEM((384,), jnp.int32),
            pltpu.VMEM((CHUNK,), jnp.int32),
            pltpu.VMEM((CHUNK,), jnp.int32),
            pltpu.VMEM((CHUNK,), jnp.int32),
            pltpu.VMEM((CHUNK,), jnp.int32),
            pltpu.VMEM((CHUNK,), jnp.float32),
            pltpu.VMEM((CHUNK,), jnp.float32),
            pltpu.VMEM((CHUNK, EMBED_DIM), jnp.float32),
            pltpu.VMEM((CHUNK, EMBED_DIM), jnp.float32),
            pltpu.SemaphoreType.DMA,
            pltpu.SemaphoreType.DMA,
            pltpu.SemaphoreType.DMA,
            pltpu.SemaphoreType.DMA,
        ],
    )
    def layer(emb_hbm, edges_hbm, zeros_hbm, out_hbm, acc,
              ebuf0, ebuf1, srcv0, srcv1, dstl0, dstl1, wb0, wb1,
              rows0, rows1, esem0, esem1, gsem0, gsem1):
        c = lax.axis_index("c")
        s = lax.axis_index("s")
        ebuf = (ebuf0, ebuf1)
        srcv = (srcv0, srcv1)
        dstl = (dstl0, dstl1)
        wb = (wb0, wb1)
        rows = (rows0, rows1)
        esem = (esem0, esem1)
        gsem = (gsem0, gsem1)

        node_base = c * HALF
        tile_edge_base = s * EDGES_PER_TILE

        zr = s * ROWS_PER_TILE
        pltpu.sync_copy(zeros_hbm.at[pl.ds(zr, ROWS_PER_TILE)],
                        acc.at[pl.ds(zr, ROWS_PER_TILE)])

        def edge_flat_base(ic):
            return (tile_edge_base + ic * CHUNK) * 3

        def extract(b):
            for g in range(CHUNK // 16):
                base = g * 48
                s16 = plsc.load_gather(ebuf[b], [_COL3 + base])
                d16 = plsc.load_gather(ebuf[b], [_COL3 + (base + 1)])
                wbits = plsc.load_gather(ebuf[b], [_COL3 + (base + 2)])
                srcv[b][pl.ds(g * 16, 16)] = s16
                dl = d16 - node_base
                ok = (dl >= 0) & (dl < HALF)
                dstl[b][pl.ds(g * 16, 16)] = jnp.where(ok, dl, HALF)
                wb[b][pl.ds(g * 16, 16)] = plsc.bitcast(wbits, jnp.float32)

        def scale(b):
            for r in range(CHUNK):
                wspl = plsc.load_gather(wb[b], [jnp.full((16,), r, jnp.int32)])
                for cc in range(4):
                    sl = pl.ds(cc * 16, 16)
                    rows[b][r, sl] = rows[b][r, sl] * wspl

        for b in (0, 1):
            pltpu.sync_copy(edges_hbm.at[pl.ds(edge_flat_base(b), 384)],
                            ebuf[b])
            extract(b)
            pltpu.async_copy(emb_hbm.at[srcv[b]], rows[b], gsem[b])

        def body(i, carry):
            for b in (0, 1):
                ic = 2 * i + b
                nxt = ic + 2

                @pl.when(nxt < NCHUNK)
                def _start_edges():
                    pltpu.async_copy(
                        edges_hbm.at[pl.ds(edge_flat_base(nxt), 384)],
                        ebuf[b], esem[b])

                pltpu.make_async_copy(emb_hbm.at[srcv[b]], rows[b],
                                      gsem[b]).wait()
                scale(b)
                pltpu.sync_copy(rows[b], acc.at[dstl[b]], add=True)

                @pl.when(nxt < NCHUNK)
                def _refill():
                    pltpu.make_async_copy(
                        edges_hbm.at[pl.ds(edge_flat_base(nxt), 384)],
                        ebuf[b], esem[b]).wait()
                    extract(b)
                    pltpu.async_copy(emb_hbm.at[srcv[b]], rows[b], gsem[b])

            return carry

        lax.fori_loop(0, NCHUNK // 2, body, 0)

        plsc.subcore_barrier()

        fb = s * ROWS_PER_TILE
        ob = c * HALF + fb

        @pl.when(s < 15)
        def _flush_full():
            pltpu.sync_copy(acc.at[pl.ds(fb, ROWS_PER_TILE)],
                            out_hbm.at[pl.ds(ob, ROWS_PER_TILE)])

        @pl.when(s == 15)
        def _flush_tail():
            pltpu.sync_copy(acc.at[pl.ds(fb, HALF - 15 * ROWS_PER_TILE)],
                            out_hbm.at[pl.ds(ob, HALF - 15 * ROWS_PER_TILE)])

    return layer(emb, edges_flat, zeros_acc)


def _mean4(e0, e1, e2, e3):
    a0 = e0.reshape(NUM_NODES // 2, 128)
    a1 = e1.reshape(NUM_NODES // 2, 128)
    a2 = e2.reshape(NUM_NODES // 2, 128)
    a3 = e3.reshape(NUM_NODES // 2, 128)

    def body(r0, r1, r2, r3, o):
        o[...] = (r0[...] + r1[...] + r2[...] + r3[...]) * 0.25

    spec = pl.BlockSpec((1000, 128), lambda i: (i, 0))
    out = pl.pallas_call(
        body,
        grid=(NUM_NODES // 2 // 1000,),
        in_specs=[spec, spec, spec, spec],
        out_specs=spec,
        out_shape=jax.ShapeDtypeStruct((NUM_NODES // 2, 128), jnp.float32),
    )(a0, a1, a2, a3)
    return out.reshape(NUM_NODES, EMBED_DIM)


def kernel(user_emb, item_emb, edge_weight, edge_index):
    e0 = jnp.concatenate([user_emb, item_emb], axis=0)

    pad = E_PAD - NUM_EDGES
    src = jnp.concatenate([edge_index[0], jnp.zeros((pad,), jnp.int32)])
    dst = jnp.concatenate([edge_index[1], jnp.zeros((pad,), jnp.int32)])
    w = jnp.concatenate([edge_weight, jnp.zeros((pad,), jnp.float32)])
    wbits = lax.bitcast_convert_type(w, jnp.int32)
    edges_flat = jnp.stack([src, dst, wbits], axis=1).reshape(-1)

    zeros_acc = jnp.zeros((ACC_ROWS, EMBED_DIM), jnp.float32)

    e1 = _propagate_layer(e0, edges_flat, zeros_acc)
    e2 = _propagate_layer(e1, edges_flat, zeros_acc)
    e3 = _propagate_layer(e2, edges_flat, zeros_acc)

    final = _mean4(e0, e1, e2, e3)
    return (final[:NUM_USERS], final[NUM_USERS:])


# speedup vs baseline: 3.7183x; 3.7183x over previous
import functools

import numpy as np

import jax
import jax.numpy as jnp
from jax import lax
from jax.experimental import pallas as pl
from jax.experimental.pallas import tpu as pltpu
from jax.experimental.pallas import tpu_sc as plsc

NUM_USERS = 25000
NUM_ITEMS = 25000
NUM_NODES = NUM_USERS + NUM_ITEMS
EMBED_DIM = 64
NUM_EDGES = 800000
NUM_LAYERS = 3

HALF = 25000
ACC_ROWS = 25088
ROWS_PER_TILE = ACC_ROWS // 16
CHUNK = 128
NCHUNK = 392
EDGES_PER_TILE = CHUNK * NCHUNK
E_PAD = EDGES_PER_TILE * 16

_COL3 = np.arange(0, 48, 3, dtype=np.int32)


def _propagate_layer(emb, edges_flat, zeros_acc):
    mesh = plsc.VectorSubcoreMesh(core_axis_name="c", subcore_axis_name="s")

    @functools.partial(
        pl.kernel,
        mesh=mesh,
        compiler_params=pltpu.CompilerParams(needs_layout_passes=False,
                                             use_tc_tiling_on_sc=False),
        out_type=jax.ShapeDtypeStruct((NUM_NODES, EMBED_DIM), jnp.float32),
        scratch_types=[
            pltpu.VMEM_SHARED((ACC_ROWS, EMBED_DIM), jnp.float32),
            pltpu.VMEM((384,), jnp.int32),
            pltpu.VMEM((384,), jnp.int32),
            pltpu.VMEM((CHUNK,), jnp.int32),
            pltpu.VMEM((CHUNK,), jnp.int32),
            pltpu.VMEM((CHUNK // 16, 16), jnp.int32),
            pltpu.VMEM((CHUNK // 16, 16), jnp.int32),
            pltpu.VMEM((CHUNK,), jnp.float32),
            pltpu.VMEM((CHUNK,), jnp.float32),
            pltpu.VMEM((CHUNK, EMBED_DIM), jnp.float32),
            pltpu.VMEM((CHUNK, EMBED_DIM), jnp.float32),
            pltpu.SemaphoreType.DMA,
            pltpu.SemaphoreType.DMA,
            pltpu.SemaphoreType.DMA,
            pltpu.SemaphoreType.DMA,
        ],
    )
    def layer(emb_hbm, edges_hbm, zeros_hbm, out_hbm, acc,
              ebuf0, ebuf1, srcv0, srcv1, dstl0, dstl1, wb0, wb1,
              rows0, rows1, esem0, esem1, gsem0, gsem1):
        c = lax.axis_index("c")
        s = lax.axis_index("s")
        ebuf = (ebuf0, ebuf1)
        srcv = (srcv0, srcv1)
        dstl = (dstl0, dstl1)
        wb = (wb0, wb1)
        rows = (rows0, rows1)
        esem = (esem0, esem1)
        gsem = (gsem0, gsem1)

        node_base = c * HALF
        tile_edge_base = s * EDGES_PER_TILE

        zr = s * ROWS_PER_TILE
        pltpu.sync_copy(zeros_hbm.at[pl.ds(zr, ROWS_PER_TILE)],
                        acc.at[pl.ds(zr, ROWS_PER_TILE)])
        plsc.subcore_barrier()

        def edge_flat_base(ic):
            return (tile_edge_base + ic * CHUNK) * 3

        def extract(b):
            col3 = lax.iota(jnp.int32, 16) * 3
            for g in range(CHUNK // 16):
                base = g * 48
                s16 = plsc.load_gather(ebuf[b], [col3 + base])
                d16 = plsc.load_gather(ebuf[b], [col3 + (base + 1)])
                wbits = plsc.load_gather(ebuf[b], [col3 + (base + 2)])
                srcv[b][pl.ds(g * 16, 16)] = s16
                dl = d16 - node_base
                ok = (dl >= 0) & (dl < HALF)
                dstl[b][g, :] = jnp.where(ok, dl, HALF)
                wb[b][pl.ds(g * 16, 16)] = plsc.bitcast(wbits, jnp.float32)

        def scale(b):
            for g in range(CHUNK // 16):
                w16 = wb[b][pl.ds(g * 16, 16)]
                for e in range(16):
                    wspl = lax.gather(
                        w16,
                        jnp.full((16, 1), e, jnp.int32),
                        lax.GatherDimensionNumbers(
                            offset_dims=(), collapsed_slice_dims=(0,),
                            start_index_map=(0,)),
                        slice_sizes=(1,),
                        mode=lax.GatherScatterMode.PROMISE_IN_BOUNDS)
                    r = g * 16 + e
                    for cc in range(4):
                        sl = pl.ds(cc * 16, 16)
                        rows[b][r, sl] = rows[b][r, sl] * wspl

        for b in (0, 1):
            pltpu.sync_copy(edges_hbm.at[pl.ds(edge_flat_base(b), 384)],
                            ebuf[b])
            extract(b)
            pltpu.async_copy(emb_hbm.at[srcv[b]], rows[b], gsem[b])

        def body(i, carry):
            for b in (0, 1):
                ic = 2 * i + b
                nxt = ic + 2

                @pl.when(nxt < NCHUNK)
                def _start_edges():
                    pltpu.async_copy(
                        edges_hbm.at[pl.ds(edge_flat_base(nxt), 384)],
                        ebuf[b], esem[b])

                pltpu.make_async_copy(emb_hbm.at[srcv[b]], rows[b],
                                      gsem[b]).wait()
                scale(b)
                for g in range(CHUNK // 16):
                    pltpu.sync_copy(rows[b].at[pl.ds(g * 16, 16)],
                                    acc.at[dstl[b].at[g]], add=True)

                @pl.when(nxt < NCHUNK)
                def _refill():
                    pltpu.make_async_copy(
                        edges_hbm.at[pl.ds(edge_flat_base(nxt), 384)],
                        ebuf[b], esem[b]).wait()
                    extract(b)
                    pltpu.async_copy(emb_hbm.at[srcv[b]], rows[b], gsem[b])

            return carry

        lax.fori_loop(0, NCHUNK // 2, body, 0)

        plsc.subcore_barrier()

        fb = s * ROWS_PER_TILE
        ob = c * HALF + fb

        @pl.when(s < 15)
        def _flush_full():
            pltpu.sync_copy(acc.at[pl.ds(fb, ROWS_PER_TILE)],
                            out_hbm.at[pl.ds(ob, ROWS_PER_TILE)])

        @pl.when(s == 15)
        def _flush_tail():
            pltpu.sync_copy(acc.at[pl.ds(fb, HALF - 15 * ROWS_PER_TILE)],
                            out_hbm.at[pl.ds(ob, HALF - 15 * ROWS_PER_TILE)])

    return layer(emb, edges_flat, zeros_acc)


def _mean4(e0, e1, e2, e3):
    a0 = e0.reshape(NUM_NODES // 2, 128)
    a1 = e1.reshape(NUM_NODES // 2, 128)
    a2 = e2.reshape(NUM_NODES // 2, 128)
    a3 = e3.reshape(NUM_NODES // 2, 128)

    def body(r0, r1, r2, r3, o):
        o[...] = (r0[...] + r1[...] + r2[...] + r3[...]) * 0.25

    spec = pl.BlockSpec((1000, 128), lambda i: (i, 0))
    out = pl.pallas_call(
        body,
        grid=(NUM_NODES // 2 // 1000,),
        in_specs=[spec, spec, spec, spec],
        out_specs=spec,
        out_shape=jax.ShapeDtypeStruct((NUM_NODES // 2, 128), jnp.float32),
    )(a0, a1, a2, a3)
    return out.reshape(NUM_NODES, EMBED_DIM)


def kernel(user_emb, item_emb, edge_weight, edge_index):
    e0 = jnp.concatenate([user_emb, item_emb], axis=0)

    pad = E_PAD - NUM_EDGES
    src = jnp.concatenate([edge_index[0], jnp.zeros((pad,), jnp.int32)])
    dst = jnp.concatenate([edge_index[1], jnp.zeros((pad,), jnp.int32)])
    w = jnp.concatenate([edge_weight, jnp.zeros((pad,), jnp.float32)])
    wbits = lax.bitcast_convert_type(w, jnp.int32)
    edges_flat = jnp.stack([src, dst, wbits], axis=1).reshape(-1)

    zeros_acc = jnp.zeros((ACC_ROWS, EMBED_DIM), jnp.float32)

    e1 = _propagate_layer(e0, edges_flat, zeros_acc)
    e2 = _propagate_layer(e1, edges_flat, zeros_acc)
    e3 = _propagate_layer(e2, edges_flat, zeros_acc)

    final = _mean4(e0, e1, e2, e3)
    return (final[:NUM_USERS], final[NUM_USERS:])


# speedup vs baseline: 3.7284x; 1.0027x over previous
import functools

import numpy as np

import jax
import jax.numpy as jnp
from jax import lax
from jax.experimental import pallas as pl
from jax.experimental.pallas import tpu as pltpu
from jax.experimental.pallas import tpu_sc as plsc

NUM_USERS = 25000
NUM_ITEMS = 25000
NUM_NODES = NUM_USERS + NUM_ITEMS
EMBED_DIM = 64
NUM_EDGES = 800000
NUM_LAYERS = 3

HALF = 25000
ACC_ROWS = 25088
ROWS_PER_TILE = ACC_ROWS // 16
CHUNK = 128
NCHUNK = 392
EDGES_PER_TILE = CHUNK * NCHUNK
E_PAD = EDGES_PER_TILE * 16

_COL3 = np.arange(0, 48, 3, dtype=np.int32)


def _propagate_layer(emb, edges_flat, zeros_acc):
    mesh = plsc.VectorSubcoreMesh(core_axis_name="c", subcore_axis_name="s")

    @functools.partial(
        pl.kernel,
        mesh=mesh,
        compiler_params=pltpu.CompilerParams(needs_layout_passes=False,
                                             use_tc_tiling_on_sc=False),
        out_type=jax.ShapeDtypeStruct((NUM_NODES, EMBED_DIM), jnp.float32),
        scratch_types=[
            pltpu.VMEM_SHARED((ACC_ROWS, EMBED_DIM), jnp.float32),
            pltpu.VMEM((384,), jnp.int32),
            pltpu.VMEM((384,), jnp.int32),
            pltpu.VMEM((CHUNK,), jnp.int32),
            pltpu.VMEM((CHUNK,), jnp.int32),
            pltpu.VMEM((CHUNK,), jnp.int32),
            pltpu.VMEM((CHUNK,), jnp.int32),
            pltpu.VMEM((CHUNK,), jnp.float32),
            pltpu.VMEM((CHUNK,), jnp.float32),
            pltpu.VMEM((CHUNK, EMBED_DIM), jnp.float32),
            pltpu.VMEM((CHUNK, EMBED_DIM), jnp.float32),
            pltpu.SemaphoreType.DMA,
            pltpu.SemaphoreType.DMA,
            pltpu.SemaphoreType.DMA,
            pltpu.SemaphoreType.DMA,
        ],
    )
    def layer(emb_hbm, edges_hbm, zeros_hbm, out_hbm, acc,
              ebuf0, ebuf1, srcv0, srcv1, dstl0, dstl1, wb0, wb1,
              rows0, rows1, esem0, esem1, gsem0, gsem1):
        c = lax.axis_index("c")
        s = lax.axis_index("s")
        ebuf = (ebuf0, ebuf1)
        srcv = (srcv0, srcv1)
        dstl = (dstl0, dstl1)
        wb = (wb0, wb1)
        rows = (rows0, rows1)
        esem = (esem0, esem1)
        gsem = (gsem0, gsem1)

        node_base = c * HALF
        tile_edge_base = s * EDGES_PER_TILE

        zr = s * ROWS_PER_TILE
        pltpu.sync_copy(zeros_hbm.at[pl.ds(zr, ROWS_PER_TILE)],
                        acc.at[pl.ds(zr, ROWS_PER_TILE)])
        plsc.subcore_barrier()

        def edge_flat_base(ic):
            return (tile_edge_base + ic * CHUNK) * 3

        def extract(b):
            col3 = lax.iota(jnp.int32, 16) * 3
            for g in range(CHUNK // 16):
                base = g * 48
                s16 = plsc.load_gather(ebuf[b], [col3 + base])
                d16 = plsc.load_gather(ebuf[b], [col3 + (base + 1)])
                wbits = plsc.load_gather(ebuf[b], [col3 + (base + 2)])
                srcv[b][pl.ds(g * 16, 16)] = s16
                dl = d16 - node_base
                ok = (dl >= 0) & (dl < HALF)
                dstl[b][pl.ds(g * 16, 16)] = jnp.where(ok, dl, HALF)
                wb[b][pl.ds(g * 16, 16)] = plsc.bitcast(wbits, jnp.float32)

        def scale(b):
            for g in range(CHUNK // 16):
                w16 = wb[b][pl.ds(g * 16, 16)]
                for e in range(16):
                    wspl = lax.gather(
                        w16,
                        jnp.full((16, 1), e, jnp.int32),
                        lax.GatherDimensionNumbers(
                            offset_dims=(), collapsed_slice_dims=(0,),
                            start_index_map=(0,)),
                        slice_sizes=(1,),
                        mode=lax.GatherScatterMode.PROMISE_IN_BOUNDS)
                    r = g * 16 + e
                    for cc in range(4):
                        sl = pl.ds(cc * 16, 16)
                        rows[b][r, sl] = rows[b][r, sl] * wspl

        for b in (0, 1):
            pltpu.sync_copy(edges_hbm.at[pl.ds(edge_flat_base(b), 384)],
                            ebuf[b])
            extract(b)
            pltpu.async_copy(emb_hbm.at[srcv[b]], rows[b], gsem[b])

        def body(i, carry):
            for b in (0, 1):
                ic = 2 * i + b
                nxt = ic + 2

                @pl.when(nxt < NCHUNK)
                def _start_edges():
                    pltpu.async_copy(
                        edges_hbm.at[pl.ds(edge_flat_base(nxt), 384)],
                        ebuf[b], esem[b])

                pltpu.make_async_copy(emb_hbm.at[srcv[b]], rows[b],
                                      gsem[b]).wait()
                scale(b)
                pltpu.sync_copy(rows[b], acc.at[dstl[b]], add=True)

                @pl.when(nxt < NCHUNK)
                def _refill():
                    pltpu.make_async_copy(
                        edges_hbm.at[pl.ds(edge_flat_base(nxt), 384)],
                        ebuf[b], esem[b]).wait()
                    extract(b)
                    pltpu.async_copy(emb_hbm.at[srcv[b]], rows[b], gsem[b])

            return carry

        lax.fori_loop(0, NCHUNK // 2, body, 0)

        plsc.subcore_barrier()

        fb = s * ROWS_PER_TILE
        ob = c * HALF + fb

        @pl.when(s < 15)
        def _flush_full():
            pltpu.sync_copy(acc.at[pl.ds(fb, ROWS_PER_TILE)],
                            out_hbm.at[pl.ds(ob, ROWS_PER_TILE)])

        @pl.when(s == 15)
        def _flush_tail():
            pltpu.sync_copy(acc.at[pl.ds(fb, HALF - 15 * ROWS_PER_TILE)],
                            out_hbm.at[pl.ds(ob, HALF - 15 * ROWS_PER_TILE)])

    return layer(emb, edges_flat, zeros_acc)


def _mean4(e0, e1, e2, e3):
    a0 = e0.reshape(NUM_NODES // 2, 128)
    a1 = e1.reshape(NUM_NODES // 2, 128)
    a2 = e2.reshape(NUM_NODES // 2, 128)
    a3 = e3.reshape(NUM_NODES // 2, 128)

    def body(r0, r1, r2, r3, o):
        o[...] = (r0[...] + r1[...] + r2[...] + r3[...]) * 0.25

    spec = pl.BlockSpec((1000, 128), lambda i: (i, 0))
    out = pl.pallas_call(
        body,
        grid=(NUM_NODES // 2 // 1000,),
        in_specs=[spec, spec, spec, spec],
        out_specs=spec,
        out_shape=jax.ShapeDtypeStruct((NUM_NODES // 2, 128), jnp.float32),
    )(a0, a1, a2, a3)
    return out.reshape(NUM_NODES, EMBED_DIM)


def kernel(user_emb, item_emb, edge_weight, edge_index):
    e0 = jnp.concatenate([user_emb, item_emb], axis=0)

    pad = E_PAD - NUM_EDGES
    src = jnp.concatenate([edge_index[0], jnp.zeros((pad,), jnp.int32)])
    dst = jnp.concatenate([edge_index[1], jnp.zeros((pad,), jnp.int32)])
    w = jnp.concatenate([edge_weight, jnp.zeros((pad,), jnp.float32)])
    wbits = lax.bitcast_convert_type(w, jnp.int32)
    edges_flat = jnp.stack([src, dst, wbits], axis=1).reshape(-1)

    zeros_acc = jnp.zeros((ACC_ROWS, EMBED_DIM), jnp.float32)

    e1 = _propagate_layer(e0, edges_flat, zeros_acc)
    e2 = _propagate_layer(e1, edges_flat, zeros_acc)
    e3 = _propagate_layer(e2, edges_flat, zeros_acc)

    final = _mean4(e0, e1, e2, e3)
    return (final[:NUM_USERS], final[NUM_USERS:])


# speedup vs baseline: 5.6314x; 1.5104x over previous
import functools

import jax
import jax.numpy as jnp
from jax import lax
from jax.experimental import pallas as pl
from jax.experimental.pallas import tpu as pltpu
from jax.experimental.pallas import tpu_sc as plsc

NUM_USERS = 25000
NUM_ITEMS = 25000
NUM_NODES = NUM_USERS + NUM_ITEMS
EMBED_DIM = 64
NUM_EDGES = 800000
NUM_LAYERS = 3

HALF = 25000
ACC_ROWS = 25088
ROWS_PER_TILE = ACC_ROWS // 16
CHUNK = 128
NCHUNK = 392
EDGES_PER_TILE = CHUNK * NCHUNK
E_PAD = EDGES_PER_TILE * 16


def _propagate_layer(emb, src, dst, w, zeros_acc):
    mesh = plsc.VectorSubcoreMesh(core_axis_name="c", subcore_axis_name="s")

    @functools.partial(
        pl.kernel,
        mesh=mesh,
        compiler_params=pltpu.CompilerParams(needs_layout_passes=False,
                                             use_tc_tiling_on_sc=False),
        out_type=jax.ShapeDtypeStruct((NUM_NODES, EMBED_DIM), jnp.float32),
        scratch_types=[
            pltpu.VMEM_SHARED((ACC_ROWS, EMBED_DIM), jnp.float32),
            pltpu.VMEM((CHUNK,), jnp.int32),
            pltpu.VMEM((CHUNK,), jnp.int32),
            pltpu.VMEM((CHUNK,), jnp.int32),
            pltpu.VMEM((CHUNK,), jnp.int32),
            pltpu.VMEM((CHUNK,), jnp.float32),
            pltpu.VMEM((CHUNK,), jnp.float32),
            pltpu.VMEM((CHUNK,), jnp.int32),
            pltpu.VMEM((CHUNK,), jnp.int32),
            pltpu.VMEM((CHUNK, EMBED_DIM), jnp.float32),
            pltpu.VMEM((CHUNK, EMBED_DIM), jnp.float32),
            pltpu.SemaphoreType.DMA,
            pltpu.SemaphoreType.DMA,
            pltpu.SemaphoreType.DMA,
            pltpu.SemaphoreType.DMA,
        ],
    )
    def layer(emb_hbm, src_hbm, dst_hbm, w_hbm, zeros_hbm, out_hbm, acc,
              sbuf0, sbuf1, dbuf0, dbuf1, wbuf0, wbuf1, dstl0, dstl1,
              rows0, rows1, esem0, esem1, gsem0, gsem1):
        c = lax.axis_index("c")
        s = lax.axis_index("s")
        sbuf = (sbuf0, sbuf1)
        dbuf = (dbuf0, dbuf1)
        wbuf = (wbuf0, wbuf1)
        dstl = (dstl0, dstl1)
        rows = (rows0, rows1)
        esem = (esem0, esem1)
        gsem = (gsem0, gsem1)

        node_base = c * HALF
        tile_edge_base = s * EDGES_PER_TILE

        zr = s * ROWS_PER_TILE
        pltpu.sync_copy(zeros_hbm.at[pl.ds(zr, ROWS_PER_TILE)],
                        acc.at[pl.ds(zr, ROWS_PER_TILE)])
        plsc.subcore_barrier()

        def start_edges(b, ic):
            eb = tile_edge_base + ic * CHUNK
            pltpu.async_copy(src_hbm.at[pl.ds(eb, CHUNK)], sbuf[b], esem[b])
            pltpu.async_copy(dst_hbm.at[pl.ds(eb, CHUNK)], dbuf[b], esem[b])
            pltpu.async_copy(w_hbm.at[pl.ds(eb, CHUNK)], wbuf[b], esem[b])

        def wait_edges(b, ic):
            eb = tile_edge_base + ic * CHUNK
            pltpu.make_async_copy(src_hbm.at[pl.ds(eb, CHUNK)], sbuf[b],
                                  esem[b]).wait()
            pltpu.make_async_copy(dst_hbm.at[pl.ds(eb, CHUNK)], dbuf[b],
                                  esem[b]).wait()
            pltpu.make_async_copy(w_hbm.at[pl.ds(eb, CHUNK)], wbuf[b],
                                  esem[b]).wait()

        def localize(b):
            for g in range(CHUNK // 16):
                sl = pl.ds(g * 16, 16)
                dl = dbuf[b][sl] - node_base
                ok = (dl >= 0) & (dl < HALF)
                dstl[b][sl] = jnp.where(ok, dl, HALF)

        def scale(b):
            for g in range(CHUNK // 16):
                w16 = wbuf[b][pl.ds(g * 16, 16)]
                for e in range(16):
                    wspl = lax.gather(
                        w16,
                        jnp.full((16, 1), e, jnp.int32),
                        lax.GatherDimensionNumbers(
                            offset_dims=(), collapsed_slice_dims=(0,),
                            start_index_map=(0,)),
                        slice_sizes=(1,),
                        mode=lax.GatherScatterMode.PROMISE_IN_BOUNDS)
                    r = g * 16 + e
                    for cc in range(4):
                        sl = pl.ds(cc * 16, 16)
                        rows[b][r, sl] = rows[b][r, sl] * wspl

        for b in (0, 1):
            start_edges(b, b)
            wait_edges(b, b)
            pltpu.async_copy(emb_hbm.at[sbuf[b]], rows[b], gsem[b])

        def body(i, carry):
            for b in (0, 1):
                ic = 2 * i + b
                nxt = ic + 2

                pltpu.make_async_copy(emb_hbm.at[sbuf[b]], rows[b],
                                      gsem[b]).wait()
                localize(b)
                scale(b)

                @pl.when(nxt < NCHUNK)
                def _start():
                    start_edges(b, nxt)

                pltpu.sync_copy(rows[b], acc.at[dstl[b]], add=True)

                @pl.when(nxt < NCHUNK)
                def _refill():
                    wait_edges(b, nxt)
                    pltpu.async_copy(emb_hbm.at[sbuf[b]], rows[b], gsem[b])

            return carry

        lax.fori_loop(0, NCHUNK // 2, body, 0)

        plsc.subcore_barrier()

        fb = s * ROWS_PER_TILE
        ob = c * HALF + fb

        @pl.when(s < 15)
        def _flush_full():
            pltpu.sync_copy(acc.at[pl.ds(fb, ROWS_PER_TILE)],
                            out_hbm.at[pl.ds(ob, ROWS_PER_TILE)])

        @pl.when(s == 15)
        def _flush_tail():
            pltpu.sync_copy(acc.at[pl.ds(fb, HALF - 15 * ROWS_PER_TILE)],
                            out_hbm.at[pl.ds(ob, HALF - 15 * ROWS_PER_TILE)])

    return layer(emb, src, dst, w, zeros_acc)


def _mean4(e0, e1, e2, e3):
    a0 = e0.reshape(NUM_NODES // 2, 128)
    a1 = e1.reshape(NUM_NODES // 2, 128)
    a2 = e2.reshape(NUM_NODES // 2, 128)
    a3 = e3.reshape(NUM_NODES // 2, 128)

    def body(r0, r1, r2, r3, o):
        o[...] = (r0[...] + r1[...] + r2[...] + r3[...]) * 0.25

    spec = pl.BlockSpec((1000, 128), lambda i: (i, 0))
    out = pl.pallas_call(
        body,
        grid=(NUM_NODES // 2 // 1000,),
        in_specs=[spec, spec, spec, spec],
        out_specs=spec,
        out_shape=jax.ShapeDtypeStruct((NUM_NODES // 2, 128), jnp.float32),
    )(a0, a1, a2, a3)
    return out.reshape(NUM_NODES, EMBED_DIM)


def kernel(user_emb, item_emb, edge_weight, edge_index):
    e0 = jnp.concatenate([user_emb, item_emb], axis=0)

    pad = E_PAD - NUM_EDGES
    src = jnp.concatenate([edge_index[0], jnp.zeros((pad,), jnp.int32)])
    dst = jnp.concatenate([edge_index[1], jnp.zeros((pad,), jnp.int32)])
    w = jnp.concatenate([edge_weight, jnp.zeros((pad,), jnp.float32)])

    zeros_acc = jnp.zeros((ACC_ROWS, EMBED_DIM), jnp.float32)

    e1 = _propagate_layer(e0, src, dst, w, zeros_acc)
    e2 = _propagate_layer(e1, src, dst, w, zeros_acc)
    e3 = _propagate_layer(e2, src, dst, w, zeros_acc)

    final = _mean4(e0, e1, e2, e3)
    return (final[:NUM_USERS], final[NUM_USERS:])
